# baseline (speedup 1.0000x reference)
import jax
import jax.numpy as jnp
from jax import lax
from jax.experimental import pallas as pl
from jax.experimental.pallas import tpu as pltpu
from jax.experimental.pallas import tpu_sc as plsc

P = 50000
U = 10000
E = 600000
D = 128

NC = 2
NS = 16
NW = NC * NS
CH = 128
NCHUNK = 152
A_SUP = 4
A_CPS = NCHUNK // A_SUP
EPW = NCHUNK * CH
EPAD = NW * EPW

A_SL = 624
A_LAST = U - (NS - 1) * A_SL

C_CHUNKS = EPAD // (NS * CH)
C_SUP = 8
C_CPS = C_CHUNKS // C_SUP
C_R = 3
C_RNG = 8336
C_ACC = 8448
C_ZPT = C_ACC // NS

_mesh = lambda: plsc.VectorSubcoreMesh(core_axis_name="c", subcore_axis_name="s",
                                       num_cores=NC, num_subcores=NS)


def _scale_chunk(rowbuf, vals_v, b, j):
  @pl.loop(0, CH, step=16)
  def _(i0):
    vv = vals_v[j, pl.ds(i0, 16)]
    for u in range(16):
      v = vv[u]
      for g in range(8):
        sl = pl.ds(g * 16, 16)
        rowbuf[b, i0 + u, sl] = rowbuf[b, i0 + u, sl] * v


def _drain(hbm_ref, rowbuf, b, sem):
  pltpu.make_async_copy(hbm_ref.at[pl.ds(0, CH)], rowbuf.at[b], sem).wait()


def _sc_a_body(geo, seq, col, upc, upr, upv, zeros, parts,
               acc, cols_v, rows_v, vals_v, rowbuf, gsem, ssem):
  c = lax.axis_index("c")
  s = lax.axis_index("s")
  w = c * NS + s

  for t, table in enumerate((geo, seq, col)):
    plsc.subcore_barrier()

    @pl.when(s < NS - 1)
    def _():
      pltpu.sync_copy(zeros.at[pl.ds(0, A_SL)], acc.at[pl.ds(s * A_SL, A_SL)])

    @pl.when(s == NS - 1)
    def _():
      pltpu.sync_copy(zeros.at[pl.ds(0, A_LAST)],
                      acc.at[pl.ds((NS - 1) * A_SL, A_LAST)])

    plsc.subcore_barrier()

    @pl.loop(0, A_SUP)
    def _(k):
      pltpu.sync_copy(upc.at[w, k], cols_v)
      pltpu.sync_copy(upr.at[w, k], rows_v)
      pltpu.sync_copy(upv.at[w, k], vals_v)

      pltpu.async_copy(table.at[cols_v.at[0]], rowbuf.at[0], gsem)

      @pl.loop(0, A_CPS, step=2)
      def _(j0):
        for b in range(2):
          j = j0 + b
          pltpu.make_async_copy(table.at[cols_v.at[j]], rowbuf.at[b],
                                gsem).wait()

          @pl.when(j + 1 < A_CPS)
          def _():
            @pl.when(j >= 1)
            def _():
              _drain(table, rowbuf, b, ssem)

            pltpu.async_copy(table.at[cols_v.at[j + 1]], rowbuf.at[1 - b],
                             gsem)

          _scale_chunk(rowbuf, vals_v, b, j)
          pltpu.async_copy(rowbuf.at[b], acc.at[rows_v.at[j]], ssem, add=True)

      for _ in range(2):
        _drain(table, rowbuf, 0, ssem)

    plsc.subcore_barrier()

    @pl.when(s < NS - 1)
    def _():
      pltpu.sync_copy(acc.at[pl.ds(s * A_SL, A_SL)],
                      parts.at[t, c, pl.ds(s * A_SL, A_SL)])

    @pl.when(s == NS - 1)
    def _():
      pltpu.sync_copy(acc.at[pl.ds((NS - 1) * A_SL, A_LAST)],
                      parts.at[t, c, pl.ds((NS - 1) * A_SL, A_LAST)])


def _sc_a(geo, seq, col, upc, upr, upv, zeros):
  return pl.kernel(
      _sc_a_body,
      out_type=jax.ShapeDtypeStruct((3, NC, U, D), jnp.float32),
      mesh=_mesh(),
      scratch_types=[
          pltpu.VMEM_SHARED((U, D), jnp.float32),
          pltpu.VMEM((A_CPS, CH), jnp.int32),
          pltpu.VMEM((A_CPS, CH), jnp.int32),
          pltpu.VMEM((A_CPS, CH), jnp.float32),
          pltpu.VMEM((2, CH, D), jnp.float32),
          pltpu.SemaphoreType.DMA,
          pltpu.SemaphoreType.DMA,
      ],
  )(geo, seq, col, upc, upr, upv, zeros)


def _sc_c_body(hg, puc, pur, puv, zeros, out,
               acc, cols_v, rows_v, vals_v, crows_v, rowbuf, gsem, ssem):
  c = lax.axis_index("c")
  s = lax.axis_index("s")

  for r in range(C_R):
    base_row = pl.multiple_of((C_R * c + r) * C_RNG, 8)
    bound = jnp.minimum(C_RNG, P - base_row)
    plsc.subcore_barrier()
    pltpu.sync_copy(zeros.at[pl.ds(0, C_ZPT)], acc.at[pl.ds(s * C_ZPT, C_ZPT)])
    plsc.subcore_barrier()

    @pl.loop(0, C_SUP)
    def _(k):
      pltpu.sync_copy(puc.at[s, k], cols_v)
      pltpu.sync_copy(pur.at[s, k], rows_v)
      pltpu.sync_copy(puv.at[s, k], vals_v)

      pltpu.async_copy(hg.at[cols_v.at[0]], rowbuf.at[0], gsem)

      @pl.loop(0, C_CPS, step=2)
      def _(j0):
        for b in range(2):
          j = j0 + b
          pltpu.make_async_copy(hg.at[cols_v.at[j]], rowbuf.at[b],
                                gsem).wait()

          @pl.when(j + 1 < C_CPS)
          def _():
            @pl.when(j >= 1)
            def _():
              _drain(hg, rowbuf, b, ssem)

            pltpu.async_copy(hg.at[cols_v.at[j + 1]], rowbuf.at[1 - b], gsem)

          for g in range(8):
            sl = pl.ds(g * 16, 16)
            lr = rows_v[j, sl] - base_row
            in_rng = (lr >= 0) & (lr < bound)
            crows_v[b, sl] = jnp.where(in_rng, lr, C_RNG)

          _scale_chunk(rowbuf, vals_v, b, j)
          pltpu.async_copy(rowbuf.at[b], acc.at[crows_v.at[b]], ssem,
                           add=True)

      for _ in range(2):
        _drain(hg, rowbuf, 0, ssem)

    plsc.subcore_barrier()

    @pl.when(s < NS - 1)
    def _():
      pltpu.sync_copy(acc.at[pl.ds(s * C_ZPT, C_ZPT)],
                      out.at[pl.ds(base_row + s * C_ZPT, C_ZPT)])

    @pl.when(s == NS - 1)
    def _():
      off = (NS - 1) * C_ZPT

      if r < C_R - 1:
        pltpu.sync_copy(acc.at[pl.ds(off, C_RNG - off)],
                        out.at[pl.ds(base_row + off, C_RNG - off)])
      else:
        @pl.when(c == 0)
        def _():
          pltpu.sync_copy(acc.at[pl.ds(off, C_RNG - off)],
                          out.at[pl.ds(base_row + off, C_RNG - off)])

        @pl.when(c == 1)
        def _():
          last = P - (2 * C_R - 1) * C_RNG - off
          pltpu.sync_copy(acc.at[pl.ds(off, last)],
                          out.at[pl.ds(base_row + off, last)])


def _sc_c(hg, puc, pur, puv, zeros):
  return pl.kernel(
      _sc_c_body,
      out_type=jax.ShapeDtypeStruct((P, D), jnp.float32),
      mesh=_mesh(),
      scratch_types=[
          pltpu.VMEM_SHARED((C_ACC, D), jnp.float32),
          pltpu.VMEM((C_CPS, CH), jnp.int32),
          pltpu.VMEM((C_CPS, CH), jnp.int32),
          pltpu.VMEM((C_CPS, CH), jnp.float32),
          pltpu.VMEM((2, CH), jnp.int32),
          pltpu.VMEM((2, CH, D), jnp.float32),
          pltpu.SemaphoreType.DMA,
          pltpu.SemaphoreType.DMA,
      ],
  )(hg, puc, pur, puv, zeros)


def _tc_b_body(p_ref, u_ref, w_ref, b_ref, o_ref):
  g = p_ref[0] + p_ref[1]
  sq = p_ref[2] + p_ref[3]
  pc = p_ref[4] + p_ref[5]
  gs = g * sq
  gp = g * pc
  sp = sq * pc
  gsp = gs * pc
  f32 = jnp.float32
  me = jnp.dot(g, w_ref[0 * D:1 * D], preferred_element_type=f32)
  me += jnp.dot(sq, w_ref[1 * D:2 * D], preferred_element_type=f32)
  me += jnp.dot(pc, w_ref[2 * D:3 * D], preferred_element_type=f32)
  me += jnp.dot(gs, w_ref[3 * D:4 * D], preferred_element_type=f32)
  me += jnp.dot(gp, w_ref[4 * D:5 * D], preferred_element_type=f32)
  me += jnp.dot(sp, w_ref[5 * D:6 * D], preferred_element_type=f32)
  me += jnp.dot(gsp, w_ref[6 * D:7 * D], preferred_element_type=f32)
  me += b_ref[...]
  usr = u_ref[...]
  o_ref[...] = me + usr + me * usr


def _tc_b(parts6, users, W, b2):
  BU = 1000
  return pl.pallas_call(
      _tc_b_body,
      out_shape=jax.ShapeDtypeStruct((U, D), jnp.float32),
      grid=(U // BU,),
      in_specs=[
          pl.BlockSpec((6, BU, D), lambda i: (0, i, 0)),
          pl.BlockSpec((BU, D), lambda i: (i, 0)),
          pl.BlockSpec((7 * D, D), lambda i: (0, 0)),
          pl.BlockSpec((1, D), lambda i: (0, 0)),
      ],
      out_specs=pl.BlockSpec((BU, D), lambda i: (i, 0)),
  )(parts6, users, W, b2)


def _pad_edges(x, fill, shape):
  pad = EPAD - E
  x = jnp.concatenate([x, jnp.full((pad,), fill, x.dtype)])
  return x.reshape(shape)


def kernel(col_pois_embs, geo_pois_embs, seq_pois_embs, users_embs,
           up_rows, up_cols, up_vals, pu_rows, pu_cols, pu_vals, W, b):
  a_shape = (NW, A_SUP, A_CPS, CH)
  c_shape = (NS, C_SUP, C_CPS, CH)
  upc = _pad_edges(up_cols, 0, a_shape)
  upr = _pad_edges(up_rows, 0, a_shape)
  upv = _pad_edges(up_vals, 0.0, a_shape)
  puc = _pad_edges(pu_cols, 0, c_shape)
  pur = _pad_edges(pu_rows, 0, c_shape)
  puv = _pad_edges(pu_vals, 0.0, c_shape)
  zeros = jnp.zeros((A_LAST, D), jnp.float32)

  parts = _sc_a(geo_pois_embs, seq_pois_embs, col_pois_embs,
                upc, upr, upv, zeros)
  hg = _tc_b(parts.reshape(6, U, D), users_embs, W, b.reshape(1, D))
  return _sc_c(hg, puc, pur, puv, zeros)


# speedup vs baseline: 2.3908x; 2.3908x over previous
import jax
import jax.numpy as jnp
from jax import lax
from jax.experimental import pallas as pl
from jax.experimental.pallas import tpu as pltpu
from jax.experimental.pallas import tpu_sc as plsc

P = 50000
U = 10000
E = 600000
D = 128

NC = 2
NS = 16
NW = NC * NS
CH = 128
NCHUNK = 147
A_SUP = 7
A_CPS = NCHUNK // A_SUP
EPW = NCHUNK * CH
EPAD = NW * EPW

A_SL = 624
A_LAST = U - (NS - 1) * A_SL

C_CHUNKS = EPAD // (NS * CH)
C_SUP = 14
C_CPS = C_CHUNKS // C_SUP
C_R = 2
C_RNG = P // C_R
C_ACC = 25088
C_ZPT = C_ACC // NS
C_LAST = C_RNG - (NS - 1) * C_ZPT
HD = D // 2

_mesh = lambda: plsc.VectorSubcoreMesh(core_axis_name="c", subcore_axis_name="s",
                                       num_cores=NC, num_subcores=NS)


def _scale_chunk(rowbuf, vals_v, b, j):
  @pl.loop(0, CH, step=16)
  def _(i0):
    vv = vals_v[j, pl.ds(i0, 16)]
    for u in range(16):
      v = vv[u]
      for g in range(8):
        sl = pl.ds(g * 16, 16)
        rowbuf[b, i0 + u, sl] = rowbuf[b, i0 + u, sl] * v


def _drain(hbm_ref, rowbuf, b, sem):
  pltpu.make_async_copy(hbm_ref.at[pl.ds(0, CH)], rowbuf.at[b], sem).wait()


def _sc_a_body(geo, seq, col, upc, upr, upv, zeros, parts,
               acc, cols_v, rows_v, vals_v, rowbuf, gsem, ssem):
  c = lax.axis_index("c")
  s = lax.axis_index("s")
  w = c * NS + s

  for t, table in enumerate((geo, seq, col)):
    plsc.subcore_barrier()

    @pl.when(s < NS - 1)
    def _():
      pltpu.sync_copy(zeros.at[pl.ds(0, A_SL)], acc.at[pl.ds(s * A_SL, A_SL)])

    @pl.when(s == NS - 1)
    def _():
      pltpu.sync_copy(zeros.at[pl.ds(0, A_LAST)],
                      acc.at[pl.ds((NS - 1) * A_SL, A_LAST)])

    plsc.subcore_barrier()

    @pl.loop(0, A_SUP)
    def _(k):
      pltpu.sync_copy(upc.at[w, k], cols_v)
      pltpu.sync_copy(upr.at[w, k], rows_v)
      pltpu.sync_copy(upv.at[w, k], vals_v)

      pltpu.async_copy(table.at[cols_v.at[0]], rowbuf.at[0], gsem)

      @pl.loop(0, A_CPS)
      def _(j):
        b = lax.rem(j, 2)
        pltpu.make_async_copy(table.at[cols_v.at[j]], rowbuf.at[b],
                              gsem).wait()

        @pl.when(j + 1 < A_CPS)
        def _():
          @pl.when(j >= 1)
          def _():
            _drain(table, rowbuf, b, ssem)

          pltpu.async_copy(table.at[cols_v.at[j + 1]], rowbuf.at[1 - b], gsem)

        _scale_chunk(rowbuf, vals_v, b, j)
        pltpu.async_copy(rowbuf.at[b], acc.at[rows_v.at[j]], ssem, add=True)

      for _ in range(2):
        _drain(table, rowbuf, 0, ssem)

    plsc.subcore_barrier()

    @pl.when(s < NS - 1)
    def _():
      pltpu.sync_copy(acc.at[pl.ds(s * A_SL, A_SL)],
                      parts.at[t, c, pl.ds(s * A_SL, A_SL)])

    @pl.when(s == NS - 1)
    def _():
      pltpu.sync_copy(acc.at[pl.ds((NS - 1) * A_SL, A_LAST)],
                      parts.at[t, c, pl.ds((NS - 1) * A_SL, A_LAST)])


def _sc_a(geo, seq, col, upc, upr, upv, zeros):
  return pl.kernel(
      _sc_a_body,
      out_type=jax.ShapeDtypeStruct((3, NC, U, D), jnp.float32),
      mesh=_mesh(),
      scratch_types=[
          pltpu.VMEM_SHARED((U, D), jnp.float32),
          pltpu.VMEM((A_CPS, CH), jnp.int32),
          pltpu.VMEM((A_CPS, CH), jnp.int32),
          pltpu.VMEM((A_CPS, CH), jnp.float32),
          pltpu.VMEM((2, CH, D), jnp.float32),
          pltpu.SemaphoreType.DMA,
          pltpu.SemaphoreType.DMA,
      ],
  )(geo, seq, col, upc, upr, upv, zeros)


def _sc_c_body(hg2, puc, pur, puv, zeros64, out,
               acc, cols_v, rows_v, vals_v, crows_v, rowbuf, gsem, ssem):
  c = lax.axis_index("c")
  s = lax.axis_index("s")

  for r in range(C_R):
    base_row = r * C_RNG
    plsc.subcore_barrier()
    pltpu.sync_copy(zeros64.at[pl.ds(0, C_ZPT)],
                    acc.at[pl.ds(s * C_ZPT, C_ZPT)])
    plsc.subcore_barrier()

    @pl.loop(0, C_SUP)
    def _(k):
      pltpu.sync_copy(puc.at[s, k], cols_v)
      pltpu.sync_copy(pur.at[s, k], rows_v)
      pltpu.sync_copy(puv.at[s, k], vals_v)

      bias = c * U

      @pl.loop(0, C_CPS)
      def _(j):
        for g in range(8):
          sl = pl.ds(g * 16, 16)
          cols_v[j, sl] = cols_v[j, sl] + bias

      pltpu.async_copy(hg2.at[cols_v.at[0]], rowbuf.at[0], gsem)

      @pl.loop(0, C_CPS)
      def _(j):
        b = lax.rem(j, 2)
        pltpu.make_async_copy(hg2.at[cols_v.at[j]], rowbuf.at[b],
                              gsem).wait()

        @pl.when(j + 1 < C_CPS)
        def _():
          @pl.when(j >= 1)
          def _():
            pltpu.make_async_copy(hg2.at[pl.ds(0, CH)], rowbuf.at[b],
                                  ssem).wait()

          pltpu.async_copy(hg2.at[cols_v.at[j + 1]], rowbuf.at[1 - b], gsem)

        for g in range(8):
          sl = pl.ds(g * 16, 16)
          lr = rows_v[j, sl] - base_row
          in_rng = (lr >= 0) & (lr < C_RNG)
          crows_v[b, sl] = jnp.where(in_rng, lr, C_RNG)

        @pl.loop(0, CH, step=16)
        def _(i0):
          vv = vals_v[j, pl.ds(i0, 16)]
          for u in range(16):
            v = vv[u]
            for g in range(4):
              sl = pl.ds(g * 16, 16)
              rowbuf[b, i0 + u, sl] = rowbuf[b, i0 + u, sl] * v

        pltpu.async_copy(rowbuf.at[b], acc.at[crows_v.at[b]], ssem, add=True)

      for _ in range(2):
        pltpu.make_async_copy(hg2.at[pl.ds(0, CH)], rowbuf.at[0], ssem).wait()

    plsc.subcore_barrier()

    @pl.when(s < NS - 1)
    def _():
      pltpu.sync_copy(acc.at[pl.ds(s * C_ZPT, C_ZPT)],
                      out.at[c, pl.ds(base_row + s * C_ZPT, C_ZPT)])

    @pl.when(s == NS - 1)
    def _():
      off = (NS - 1) * C_ZPT
      pltpu.sync_copy(acc.at[pl.ds(off, C_LAST)],
                      out.at[c, pl.ds(base_row + off, C_LAST)])


def _sc_c(hg2, puc, pur, puv, zeros64):
  return pl.kernel(
      _sc_c_body,
      out_type=jax.ShapeDtypeStruct((NC, P, HD), jnp.float32),
      mesh=_mesh(),
      compiler_params=pltpu.CompilerParams(use_tc_tiling_on_sc=False),
      scratch_types=[
          pltpu.VMEM_SHARED((C_ACC, HD), jnp.float32),
          pltpu.VMEM((C_CPS, CH), jnp.int32),
          pltpu.VMEM((C_CPS, CH), jnp.int32),
          pltpu.VMEM((C_CPS, CH), jnp.float32),
          pltpu.VMEM((2, CH), jnp.int32),
          pltpu.VMEM((2, CH, HD), jnp.float32),
          pltpu.SemaphoreType.DMA,
          pltpu.SemaphoreType.DMA,
      ],
  )(hg2, puc, pur, puv, zeros64)


def _tc_b_body(p_ref, u_ref, w_ref, b_ref, o0_ref, o1_ref):
  g = p_ref[0] + p_ref[1]
  sq = p_ref[2] + p_ref[3]
  pc = p_ref[4] + p_ref[5]
  gs = g * sq
  gp = g * pc
  sp = sq * pc
  gsp = gs * pc
  f32 = jnp.float32
  me = jnp.dot(g, w_ref[0 * D:1 * D], preferred_element_type=f32)
  me += jnp.dot(sq, w_ref[1 * D:2 * D], preferred_element_type=f32)
  me += jnp.dot(pc, w_ref[2 * D:3 * D], preferred_element_type=f32)
  me += jnp.dot(gs, w_ref[3 * D:4 * D], preferred_element_type=f32)
  me += jnp.dot(gp, w_ref[4 * D:5 * D], preferred_element_type=f32)
  me += jnp.dot(sp, w_ref[5 * D:6 * D], preferred_element_type=f32)
  me += jnp.dot(gsp, w_ref[6 * D:7 * D], preferred_element_type=f32)
  me += b_ref[...]
  usr = u_ref[...]
  hg = me + usr + me * usr
  o0_ref[...] = hg[:, :HD]
  o1_ref[...] = hg[:, HD:]


def _tc_b(parts6, users, W, b2):
  BU = 1000
  return pl.pallas_call(
      _tc_b_body,
      out_shape=[jax.ShapeDtypeStruct((U, HD), jnp.float32),
                 jax.ShapeDtypeStruct((U, HD), jnp.float32)],
      grid=(U // BU,),
      in_specs=[
          pl.BlockSpec((6, BU, D), lambda i: (0, i, 0)),
          pl.BlockSpec((BU, D), lambda i: (i, 0)),
          pl.BlockSpec((7 * D, D), lambda i: (0, 0)),
          pl.BlockSpec((1, D), lambda i: (0, 0)),
      ],
      out_specs=[pl.BlockSpec((BU, HD), lambda i: (i, 0)),
                 pl.BlockSpec((BU, HD), lambda i: (i, 0))],
  )(parts6, users, W, b2)


def _pad_edges(x, fill, shape):
  pad = EPAD - E
  x = jnp.concatenate([x, jnp.full((pad,), fill, x.dtype)])
  return x.reshape(shape)


def kernel(col_pois_embs, geo_pois_embs, seq_pois_embs, users_embs,
           up_rows, up_cols, up_vals, pu_rows, pu_cols, pu_vals, W, b):
  a_shape = (NW, A_SUP, A_CPS, CH)
  c_shape = (NS, C_SUP, C_CPS, CH)
  upc = _pad_edges(up_cols, 0, a_shape)
  upr = _pad_edges(up_rows, 0, a_shape)
  upv = _pad_edges(up_vals, 0.0, a_shape)
  puc = _pad_edges(pu_cols, 0, c_shape)
  pur = _pad_edges(pu_rows, 0, c_shape)
  puv = _pad_edges(pu_vals, 0.0, c_shape)
  zeros = jnp.zeros((A_LAST, D), jnp.float32)
  zeros64 = jnp.zeros((C_ZPT, HD), jnp.float32)

  parts = _sc_a(geo_pois_embs, seq_pois_embs, col_pois_embs,
                upc, upr, upv, zeros)
  h0, h1 = _tc_b(parts.reshape(6, U, D), users_embs, W, b.reshape(1, D))
  hg2 = jnp.concatenate([h0, h1], axis=0)
  out2 = _sc_c(hg2, puc, pur, puv, zeros64)
  return jnp.concatenate([out2[0], out2[1]], axis=1)


# speedup vs baseline: 2.4405x; 1.0208x over previous
import jax
import jax.numpy as jnp
from jax import lax
from jax.experimental import pallas as pl
from jax.experimental.pallas import tpu as pltpu
from jax.experimental.pallas import tpu_sc as plsc

P = 50000
U = 10000
E = 600000
D = 128

NC = 2
NS = 16
NW = NC * NS
CH = 128
NCHUNK = 147
A_SUP = 7
A_CPS = NCHUNK // A_SUP
EPW = NCHUNK * CH
EPAD = NW * EPW

A_SL = 624
A_LAST = U - (NS - 1) * A_SL

C_CHUNKS = EPAD // (NS * CH)
C_SUP = 14
C_CPS = C_CHUNKS // C_SUP
C_R = 2
C_RNG = P // C_R
C_ACC = 25088
C_ZPT = C_ACC // NS
C_LAST = C_RNG - (NS - 1) * C_ZPT
HD = D // 2

_mesh = lambda: plsc.VectorSubcoreMesh(core_axis_name="c", subcore_axis_name="s",
                                       num_cores=NC, num_subcores=NS)


def _scale_chunk(rowbuf, vals_v, b, j):
  @pl.loop(0, CH, step=16)
  def _(i0):
    vv = vals_v[j, pl.ds(i0, 16)]
    for u in range(16):
      v = vv[u]
      for g in range(8):
        sl = pl.ds(g * 16, 16)
        rowbuf[b, i0 + u, sl] = rowbuf[b, i0 + u, sl] * v


def _drain(hbm_ref, rowbuf, b, sem):
  pltpu.make_async_copy(hbm_ref.at[pl.ds(0, CH)], rowbuf.at[b], sem).wait()


def _sc_a_body(t0, t1, t2, upc, upr, upv, zeros64, parts,
               acc, cols_v, rows_v, vals_v, rowbuf, gsem, ssem):
  c = lax.axis_index("c")
  s = lax.axis_index("s")

  for t, table in enumerate((t0, t1, t2)):
    plsc.subcore_barrier()

    @pl.when(s < NS - 1)
    def _():
      pltpu.sync_copy(zeros64.at[pl.ds(0, A_SL)],
                      acc.at[pl.ds(s * A_SL, A_SL)])

    @pl.when(s == NS - 1)
    def _():
      pltpu.sync_copy(zeros64.at[pl.ds(0, A_LAST)],
                      acc.at[pl.ds((NS - 1) * A_SL, A_LAST)])

    plsc.subcore_barrier()

    @pl.loop(0, C_SUP)
    def _(k):
      pltpu.sync_copy(upc.at[s, k], cols_v)
      pltpu.sync_copy(upr.at[s, k], rows_v)
      pltpu.sync_copy(upv.at[s, k], vals_v)

      bias = c * P

      @pl.loop(0, C_CPS)
      def _(j):
        for g in range(8):
          sl = pl.ds(g * 16, 16)
          cols_v[j, sl] = cols_v[j, sl] + bias

      pltpu.async_copy(table.at[cols_v.at[0]], rowbuf.at[0], gsem)
      pltpu.async_copy(table.at[cols_v.at[1]], rowbuf.at[1], gsem)

      @pl.loop(0, C_CPS)
      def _(j):
        b = lax.rem(j, 4)
        pltpu.make_async_copy(table.at[cols_v.at[j]], rowbuf.at[b],
                              gsem).wait()

        @pl.when(j + 2 < C_CPS)
        def _():
          @pl.when(j >= 2)
          def _():
            pltpu.make_async_copy(table.at[pl.ds(0, CH)],
                                  rowbuf.at[b], ssem).wait()

          pltpu.async_copy(table.at[cols_v.at[j + 2]],
                           rowbuf.at[lax.rem(j + 2, 4)], gsem)

        @pl.loop(0, CH, step=16)
        def _(i0):
          vv = vals_v[j, pl.ds(i0, 16)]
          for u in range(16):
            v = vv[u]
            for g in range(4):
              sl = pl.ds(g * 16, 16)
              rowbuf[b, i0 + u, sl] = rowbuf[b, i0 + u, sl] * v

        pltpu.async_copy(rowbuf.at[b], acc.at[rows_v.at[j]], ssem, add=True)

      for _ in range(4):
        pltpu.make_async_copy(table.at[pl.ds(0, CH)], rowbuf.at[0],
                              ssem).wait()

    plsc.subcore_barrier()

    @pl.when(s < NS - 1)
    def _():
      pltpu.sync_copy(acc.at[pl.ds(s * A_SL, A_SL)],
                      parts.at[t, c, pl.ds(s * A_SL, A_SL)])

    @pl.when(s == NS - 1)
    def _():
      pltpu.sync_copy(acc.at[pl.ds((NS - 1) * A_SL, A_LAST)],
                      parts.at[t, c, pl.ds((NS - 1) * A_SL, A_LAST)])


def _sc_a(t0, t1, t2, upc, upr, upv, zeros64):
  return pl.kernel(
      _sc_a_body,
      out_type=jax.ShapeDtypeStruct((3, NC, U, HD), jnp.float32),
      mesh=_mesh(),
      compiler_params=pltpu.CompilerParams(use_tc_tiling_on_sc=False),
      scratch_types=[
          pltpu.VMEM_SHARED((U, HD), jnp.float32),
          pltpu.VMEM((C_CPS, CH), jnp.int32),
          pltpu.VMEM((C_CPS, CH), jnp.int32),
          pltpu.VMEM((C_CPS, CH), jnp.float32),
          pltpu.VMEM((4, CH, HD), jnp.float32),
          pltpu.SemaphoreType.DMA,
          pltpu.SemaphoreType.DMA,
      ],
  )(t0, t1, t2, upc, upr, upv, zeros64)


def _sc_c_body(hg2, puc, pur, puv, zeros64, out,
               acc, cols_v, rows_v, vals_v, crows_v, rowbuf, gsem, ssem):
  c = lax.axis_index("c")
  s = lax.axis_index("s")

  for r in range(C_R):
    base_row = r * C_RNG
    plsc.subcore_barrier()
    pltpu.sync_copy(zeros64.at[pl.ds(0, C_ZPT)],
                    acc.at[pl.ds(s * C_ZPT, C_ZPT)])
    plsc.subcore_barrier()

    @pl.loop(0, C_SUP)
    def _(k):
      pltpu.sync_copy(puc.at[s, k], cols_v)
      pltpu.sync_copy(pur.at[s, k], rows_v)
      pltpu.sync_copy(puv.at[s, k], vals_v)

      bias = c * U

      @pl.loop(0, C_CPS)
      def _(j):
        for g in range(8):
          sl = pl.ds(g * 16, 16)
          cols_v[j, sl] = cols_v[j, sl] + bias

      pltpu.async_copy(hg2.at[cols_v.at[0]], rowbuf.at[0], gsem)

      @pl.loop(0, C_CPS)
      def _(j):
        b = lax.rem(j, 2)
        pltpu.make_async_copy(hg2.at[cols_v.at[j]], rowbuf.at[b],
                              gsem).wait()

        @pl.when(j + 1 < C_CPS)
        def _():
          @pl.when(j >= 1)
          def _():
            pltpu.make_async_copy(hg2.at[pl.ds(0, CH)], rowbuf.at[b],
                                  ssem).wait()

          pltpu.async_copy(hg2.at[cols_v.at[j + 1]], rowbuf.at[1 - b], gsem)

        for g in range(8):
          sl = pl.ds(g * 16, 16)
          lr = rows_v[j, sl] - base_row
          in_rng = (lr >= 0) & (lr < C_RNG)
          crows_v[b, sl] = jnp.where(in_rng, lr, C_RNG)

        @pl.loop(0, CH, step=16)
        def _(i0):
          vv = vals_v[j, pl.ds(i0, 16)]
          for u in range(16):
            v = vv[u]
            for g in range(4):
              sl = pl.ds(g * 16, 16)
              rowbuf[b, i0 + u, sl] = rowbuf[b, i0 + u, sl] * v

        pltpu.async_copy(rowbuf.at[b], acc.at[crows_v.at[b]], ssem, add=True)

      for _ in range(2):
        pltpu.make_async_copy(hg2.at[pl.ds(0, CH)], rowbuf.at[0], ssem).wait()

    plsc.subcore_barrier()

    @pl.when(s < NS - 1)
    def _():
      pltpu.sync_copy(acc.at[pl.ds(s * C_ZPT, C_ZPT)],
                      out.at[c, pl.ds(base_row + s * C_ZPT, C_ZPT)])

    @pl.when(s == NS - 1)
    def _():
      off = (NS - 1) * C_ZPT
      pltpu.sync_copy(acc.at[pl.ds(off, C_LAST)],
                      out.at[c, pl.ds(base_row + off, C_LAST)])


def _sc_c(hg2, puc, pur, puv, zeros64):
  return pl.kernel(
      _sc_c_body,
      out_type=jax.ShapeDtypeStruct((NC, P, HD), jnp.float32),
      mesh=_mesh(),
      compiler_params=pltpu.CompilerParams(use_tc_tiling_on_sc=False),
      scratch_types=[
          pltpu.VMEM_SHARED((C_ACC, HD), jnp.float32),
          pltpu.VMEM((C_CPS, CH), jnp.int32),
          pltpu.VMEM((C_CPS, CH), jnp.int32),
          pltpu.VMEM((C_CPS, CH), jnp.float32),
          pltpu.VMEM((2, CH), jnp.int32),
          pltpu.VMEM((2, CH, HD), jnp.float32),
          pltpu.SemaphoreType.DMA,
          pltpu.SemaphoreType.DMA,
      ],
  )(hg2, puc, pur, puv, zeros64)


def _tc_b_body(p_ref, u_ref, w_ref, b_ref, o0_ref, o1_ref):
  g = p_ref[0]
  sq = p_ref[1]
  pc = p_ref[2]
  gs = g * sq
  gp = g * pc
  sp = sq * pc
  gsp = gs * pc
  f32 = jnp.float32
  me = jnp.dot(g, w_ref[0 * D:1 * D], preferred_element_type=f32)
  me += jnp.dot(sq, w_ref[1 * D:2 * D], preferred_element_type=f32)
  me += jnp.dot(pc, w_ref[2 * D:3 * D], preferred_element_type=f32)
  me += jnp.dot(gs, w_ref[3 * D:4 * D], preferred_element_type=f32)
  me += jnp.dot(gp, w_ref[4 * D:5 * D], preferred_element_type=f32)
  me += jnp.dot(sp, w_ref[5 * D:6 * D], preferred_element_type=f32)
  me += jnp.dot(gsp, w_ref[6 * D:7 * D], preferred_element_type=f32)
  me += b_ref[...]
  usr = u_ref[...]
  hg = me + usr + me * usr
  o0_ref[...] = hg[:, :HD]
  o1_ref[...] = hg[:, HD:]


def _tc_b(parts6, users, W, b2):
  BU = 1000
  return pl.pallas_call(
      _tc_b_body,
      out_shape=[jax.ShapeDtypeStruct((U, HD), jnp.float32),
                 jax.ShapeDtypeStruct((U, HD), jnp.float32)],
      grid=(U // BU,),
      in_specs=[
          pl.BlockSpec((3, BU, D), lambda i: (0, i, 0)),
          pl.BlockSpec((BU, D), lambda i: (i, 0)),
          pl.BlockSpec((7 * D, D), lambda i: (0, 0)),
          pl.BlockSpec((1, D), lambda i: (0, 0)),
      ],
      out_specs=[pl.BlockSpec((BU, HD), lambda i: (i, 0)),
                 pl.BlockSpec((BU, HD), lambda i: (i, 0))],
  )(parts6, users, W, b2)


def _pad_edges(x, fill, shape):
  pad = EPAD - E
  x = jnp.concatenate([x, jnp.full((pad,), fill, x.dtype)])
  return x.reshape(shape)


def kernel(col_pois_embs, geo_pois_embs, seq_pois_embs, users_embs,
           up_rows, up_cols, up_vals, pu_rows, pu_cols, pu_vals, W, b):
  c_shape = (NS, C_SUP, C_CPS, CH)
  upc = _pad_edges(up_cols, 0, c_shape)
  upr = _pad_edges(up_rows, 0, c_shape)
  upv = _pad_edges(up_vals, 0.0, c_shape)
  puc = _pad_edges(pu_cols, 0, c_shape)
  pur = _pad_edges(pu_rows, 0, c_shape)
  puv = _pad_edges(pu_vals, 0.0, c_shape)
  zeros64 = jnp.zeros((C_ZPT, HD), jnp.float32)

  stack = lambda T: jnp.concatenate([T[:, :HD], T[:, HD:]], axis=0)
  parts = _sc_a(stack(geo_pois_embs), stack(seq_pois_embs),
                stack(col_pois_embs), upc, upr, upv, zeros64)
  aggs = jnp.concatenate([parts[:, 0], parts[:, 1]], axis=2)
  h0, h1 = _tc_b(aggs, users_embs, W, b.reshape(1, D))
  hg2 = jnp.concatenate([h0, h1], axis=0)
  out2 = _sc_c(hg2, puc, pur, puv, zeros64)
  return jnp.concatenate([out2[0], out2[1]], axis=1)


# speedup vs baseline: 2.8436x; 1.1652x over previous
import jax
import jax.numpy as jnp
from jax import lax
from jax.experimental import pallas as pl
from jax.experimental.pallas import tpu as pltpu
from jax.experimental.pallas import tpu_sc as plsc

P = 50000
U = 10000
E = 600000
D = 128

NC = 2
NS = 16
NW = NC * NS
CH = 128
NCHUNK = 147
A_SUP = 7
A_CPS = NCHUNK // A_SUP
EPW = NCHUNK * CH
EPAD = NW * EPW

A_SL = 624
A_LAST = U - (NS - 1) * A_SL

C_CHUNKS = EPAD // (NS * CH)
C_SUP = 14
C_CPS = C_CHUNKS // C_SUP
C_R = 2
C_RNG = P // C_R
C_ACC = 25088
C_ZPT = C_ACC // NS
C_LAST = C_RNG - (NS - 1) * C_ZPT
HD = D // 2

_mesh = lambda: plsc.VectorSubcoreMesh(core_axis_name="c", subcore_axis_name="s",
                                       num_cores=NC, num_subcores=NS)


def _scale4(rowbuf, vals_v, b, j):
  @pl.loop(0, CH, step=16)
  def _(i0):
    vv = vals_v[j, pl.ds(i0, 16)]
    for u in range(16):
      v = vv[u]
      for g in range(4):
        sl = pl.ds(g * 16, 16)
        rowbuf[b, i0 + u, sl] = rowbuf[b, i0 + u, sl] * v


def _scale_switch(rowbuf, vals_v, b, j, nbuf):
  for bb in range(nbuf):
    @pl.when(b == bb)
    def _():
      _scale4(rowbuf, vals_v, bb, j)


def _scale_chunk(rowbuf, vals_v, b, j):
  @pl.loop(0, CH, step=16)
  def _(i0):
    vv = vals_v[j, pl.ds(i0, 16)]
    for u in range(16):
      v = vv[u]
      for g in range(8):
        sl = pl.ds(g * 16, 16)
        rowbuf[b, i0 + u, sl] = rowbuf[b, i0 + u, sl] * v


def _drain(hbm_ref, rowbuf, b, sem):
  pltpu.make_async_copy(hbm_ref.at[pl.ds(0, CH)], rowbuf.at[b], sem).wait()


def _sc_a_body(t0, t1, t2, upc, upr, upv, zeros64, parts,
               acc, cols_v, rows_v, vals_v, rowbuf, gsem, ssem):
  c = lax.axis_index("c")
  s = lax.axis_index("s")

  for t, table in enumerate((t0, t1, t2)):
    plsc.subcore_barrier()

    @pl.when(s < NS - 1)
    def _():
      pltpu.sync_copy(zeros64.at[pl.ds(0, A_SL)],
                      acc.at[pl.ds(s * A_SL, A_SL)])

    @pl.when(s == NS - 1)
    def _():
      pltpu.sync_copy(zeros64.at[pl.ds(0, A_LAST)],
                      acc.at[pl.ds((NS - 1) * A_SL, A_LAST)])

    plsc.subcore_barrier()

    @pl.loop(0, C_SUP)
    def _(k):
      pltpu.sync_copy(upc.at[s, k], cols_v)
      pltpu.sync_copy(upr.at[s, k], rows_v)
      pltpu.sync_copy(upv.at[s, k], vals_v)

      bias = c * P

      @pl.loop(0, C_CPS)
      def _(j):
        for g in range(8):
          sl = pl.ds(g * 16, 16)
          cols_v[j, sl] = cols_v[j, sl] + bias

      pltpu.async_copy(table.at[cols_v.at[0]], rowbuf.at[0], gsem)
      pltpu.async_copy(table.at[cols_v.at[1]], rowbuf.at[1], gsem)

      @pl.loop(0, C_CPS)
      def _(j):
        b = lax.rem(j, 4)
        pltpu.make_async_copy(table.at[cols_v.at[j]], rowbuf.at[b],
                              gsem).wait()

        @pl.when(j + 2 < C_CPS)
        def _():
          @pl.when(j >= 2)
          def _():
            pltpu.make_async_copy(table.at[pl.ds(0, CH)],
                                  rowbuf.at[b], ssem).wait()

          pltpu.async_copy(table.at[cols_v.at[j + 2]],
                           rowbuf.at[lax.rem(j + 2, 4)], gsem)

        _scale_switch(rowbuf, vals_v, b, j, 4)

        pltpu.async_copy(rowbuf.at[b], acc.at[rows_v.at[j]], ssem, add=True)

      for _ in range(4):
        pltpu.make_async_copy(table.at[pl.ds(0, CH)], rowbuf.at[0],
                              ssem).wait()

    plsc.subcore_barrier()

    @pl.when(s < NS - 1)
    def _():
      pltpu.sync_copy(acc.at[pl.ds(s * A_SL, A_SL)],
                      parts.at[t, c, pl.ds(s * A_SL, A_SL)])

    @pl.when(s == NS - 1)
    def _():
      pltpu.sync_copy(acc.at[pl.ds((NS - 1) * A_SL, A_LAST)],
                      parts.at[t, c, pl.ds((NS - 1) * A_SL, A_LAST)])


def _sc_a(t0, t1, t2, upc, upr, upv, zeros64):
  return pl.kernel(
      _sc_a_body,
      out_type=jax.ShapeDtypeStruct((3, NC, U, HD), jnp.float32),
      mesh=_mesh(),
      compiler_params=pltpu.CompilerParams(use_tc_tiling_on_sc=False),
      scratch_types=[
          pltpu.VMEM_SHARED((U, HD), jnp.float32),
          pltpu.VMEM((C_CPS, CH), jnp.int32),
          pltpu.VMEM((C_CPS, CH), jnp.int32),
          pltpu.VMEM((C_CPS, CH), jnp.float32),
          pltpu.VMEM((4, CH, HD), jnp.float32),
          pltpu.SemaphoreType.DMA,
          pltpu.SemaphoreType.DMA,
      ],
  )(t0, t1, t2, upc, upr, upv, zeros64)


def _sc_c_body(hg2, puc, pur, puv, zeros64, out,
               acc, cols_v, rows_v, vals_v, crows_v, rowbuf, gsem, ssem):
  c = lax.axis_index("c")
  s = lax.axis_index("s")

  for r in range(C_R):
    base_row = r * C_RNG
    plsc.subcore_barrier()
    pltpu.sync_copy(zeros64.at[pl.ds(0, C_ZPT)],
                    acc.at[pl.ds(s * C_ZPT, C_ZPT)])
    plsc.subcore_barrier()

    @pl.loop(0, C_SUP)
    def _(k):
      pltpu.sync_copy(puc.at[s, k], cols_v)
      pltpu.sync_copy(pur.at[s, k], rows_v)
      pltpu.sync_copy(puv.at[s, k], vals_v)

      bias = c * U

      @pl.loop(0, C_CPS)
      def _(j):
        for g in range(8):
          sl = pl.ds(g * 16, 16)
          cols_v[j, sl] = cols_v[j, sl] + bias

      pltpu.async_copy(hg2.at[cols_v.at[0]], rowbuf.at[0], gsem)

      @pl.loop(0, C_CPS)
      def _(j):
        b = lax.rem(j, 2)
        pltpu.make_async_copy(hg2.at[cols_v.at[j]], rowbuf.at[b],
                              gsem).wait()

        @pl.when(j + 1 < C_CPS)
        def _():
          @pl.when(j >= 1)
          def _():
            pltpu.make_async_copy(hg2.at[pl.ds(0, CH)], rowbuf.at[b],
                                  ssem).wait()

          pltpu.async_copy(hg2.at[cols_v.at[j + 1]], rowbuf.at[1 - b], gsem)

        for g in range(8):
          sl = pl.ds(g * 16, 16)
          lr = rows_v[j, sl] - base_row
          in_rng = (lr >= 0) & (lr < C_RNG)
          crows_v[b, sl] = jnp.where(in_rng, lr, C_RNG)

        _scale_switch(rowbuf, vals_v, b, j, 2)

        pltpu.async_copy(rowbuf.at[b], acc.at[crows_v.at[b]], ssem, add=True)

      for _ in range(2):
        pltpu.make_async_copy(hg2.at[pl.ds(0, CH)], rowbuf.at[0], ssem).wait()

    plsc.subcore_barrier()

    @pl.when(s < NS - 1)
    def _():
      pltpu.sync_copy(acc.at[pl.ds(s * C_ZPT, C_ZPT)],
                      out.at[c, pl.ds(base_row + s * C_ZPT, C_ZPT)])

    @pl.when(s == NS - 1)
    def _():
      off = (NS - 1) * C_ZPT
      pltpu.sync_copy(acc.at[pl.ds(off, C_LAST)],
                      out.at[c, pl.ds(base_row + off, C_LAST)])


def _sc_c(hg2, puc, pur, puv, zeros64):
  return pl.kernel(
      _sc_c_body,
      out_type=jax.ShapeDtypeStruct((NC, P, HD), jnp.float32),
      mesh=_mesh(),
      compiler_params=pltpu.CompilerParams(use_tc_tiling_on_sc=False),
      scratch_types=[
          pltpu.VMEM_SHARED((C_ACC, HD), jnp.float32),
          pltpu.VMEM((C_CPS, CH), jnp.int32),
          pltpu.VMEM((C_CPS, CH), jnp.int32),
          pltpu.VMEM((C_CPS, CH), jnp.float32),
          pltpu.VMEM((2, CH), jnp.int32),
          pltpu.VMEM((2, CH, HD), jnp.float32),
          pltpu.SemaphoreType.DMA,
          pltpu.SemaphoreType.DMA,
      ],
  )(hg2, puc, pur, puv, zeros64)


def _tc_b_body(p_ref, u_ref, w_ref, b_ref, o0_ref, o1_ref):
  g = p_ref[0]
  sq = p_ref[1]
  pc = p_ref[2]
  gs = g * sq
  gp = g * pc
  sp = sq * pc
  gsp = gs * pc
  f32 = jnp.float32
  me = jnp.dot(g, w_ref[0 * D:1 * D], preferred_element_type=f32)
  me += jnp.dot(sq, w_ref[1 * D:2 * D], preferred_element_type=f32)
  me += jnp.dot(pc, w_ref[2 * D:3 * D], preferred_element_type=f32)
  me += jnp.dot(gs, w_ref[3 * D:4 * D], preferred_element_type=f32)
  me += jnp.dot(gp, w_ref[4 * D:5 * D], preferred_element_type=f32)
  me += jnp.dot(sp, w_ref[5 * D:6 * D], preferred_element_type=f32)
  me += jnp.dot(gsp, w_ref[6 * D:7 * D], preferred_element_type=f32)
  me += b_ref[...]
  usr = u_ref[...]
  hg = me + usr + me * usr
  o0_ref[...] = hg[:, :HD]
  o1_ref[...] = hg[:, HD:]


def _tc_b(parts6, users, W, b2):
  BU = 1000
  return pl.pallas_call(
      _tc_b_body,
      out_shape=[jax.ShapeDtypeStruct((U, HD), jnp.float32),
                 jax.ShapeDtypeStruct((U, HD), jnp.float32)],
      grid=(U // BU,),
      in_specs=[
          pl.BlockSpec((3, BU, D), lambda i: (0, i, 0)),
          pl.BlockSpec((BU, D), lambda i: (i, 0)),
          pl.BlockSpec((7 * D, D), lambda i: (0, 0)),
          pl.BlockSpec((1, D), lambda i: (0, 0)),
      ],
      out_specs=[pl.BlockSpec((BU, HD), lambda i: (i, 0)),
                 pl.BlockSpec((BU, HD), lambda i: (i, 0))],
  )(parts6, users, W, b2)


def _pad_edges(x, fill, shape):
  pad = EPAD - E
  x = jnp.concatenate([x, jnp.full((pad,), fill, x.dtype)])
  return x.reshape(shape)


def kernel(col_pois_embs, geo_pois_embs, seq_pois_embs, users_embs,
           up_rows, up_cols, up_vals, pu_rows, pu_cols, pu_vals, W, b):
  c_shape = (NS, C_SUP, C_CPS, CH)
  upc = _pad_edges(up_cols, 0, c_shape)
  upr = _pad_edges(up_rows, 0, c_shape)
  upv = _pad_edges(up_vals, 0.0, c_shape)
  puc = _pad_edges(pu_cols, 0, c_shape)
  pur = _pad_edges(pu_rows, 0, c_shape)
  puv = _pad_edges(pu_vals, 0.0, c_shape)
  zeros64 = jnp.zeros((C_ZPT, HD), jnp.float32)

  stack = lambda T: jnp.concatenate([T[:, :HD], T[:, HD:]], axis=0)
  parts = _sc_a(stack(geo_pois_embs), stack(seq_pois_embs),
                stack(col_pois_embs), upc, upr, upv, zeros64)
  aggs = jnp.concatenate([parts[:, 0], parts[:, 1]], axis=2)
  h0, h1 = _tc_b(aggs, users_embs, W, b.reshape(1, D))
  hg2 = jnp.concatenate([h0, h1], axis=0)
  out2 = _sc_c(hg2, puc, pur, puv, zeros64)
  return jnp.concatenate([out2[0], out2[1]], axis=1)
